# baseline (device time: 11790 ns/iter reference)
import jax
import jax.numpy as jnp
from jax import lax
from jax.experimental import pallas as pl
from jax.experimental.pallas import tpu as pltpu

_CHUNK = 32


def kernel(x, dest):
    m, n = x.shape
    me = lax.axis_index("y")
    ns = jnp.sum((dest != me).astype(jnp.int32))
    nk = jnp.int32(m) - ns
    cnt = jnp.stack([nk, ns])

    n_chunks = m // _CHUNK

    def body(cnt_ref, dest_ref, x_ref, out_ref, xks_buf, recv_buf,
             send_sems, recv_sems):
        my_x = lax.axis_index("x")
        my_y = lax.axis_index("y")
        my_z = lax.axis_index("z")
        peer = (my_x, 1 - my_y, my_z)
        nk_ = cnt_ref[0]
        ns_ = cnt_ref[1]

        barrier_sem = pltpu.get_barrier_semaphore()
        pl.semaphore_signal(
            barrier_sem, inc=1, device_id=peer,
            device_id_type=pl.DeviceIdType.MESH,
        )
        pl.semaphore_wait(barrier_sem, 1)

        vf = (dest_ref[:, :] != my_y).astype(jnp.float32)
        lane = lax.broadcasted_iota(jnp.int32, (1, m), 1)
        cs = vf
        s = 1
        while s < m:
            cs = cs + jnp.where(lane >= s, pltpu.roll(cs, s, axis=1), 0.0)
            s *= 2
        idxr = lane.astype(jnp.float32)
        pos = jnp.where(vf == 1.0, nk_.astype(jnp.float32) + cs - 1.0, idxr - cs)

        j_f = lax.broadcasted_iota(jnp.int32, (m, m), 0).astype(jnp.float32)
        p_mat = (j_f == pos).astype(jnp.float32)

        def chunk_rdma(c):
            return pltpu.make_async_remote_copy(
                src_ref=xks_buf.at[pl.ds(c * _CHUNK, _CHUNK)],
                dst_ref=recv_buf.at[pl.ds(c * _CHUNK, _CHUNK)],
                send_sem=send_sems.at[c],
                recv_sem=recv_sems.at[c],
                device_id=peer,
                device_id_type=pl.DeviceIdType.MESH,
            )

        def pred(c):
            if c == n_chunks - 1:
                return nk_ >= 0
            return (c + 1) * _CHUNK > nk_

        for c in reversed(range(n_chunks)):
            xks_buf[c * _CHUNK:(c + 1) * _CHUNK, :] = jax.lax.dot(
                p_mat[c * _CHUNK:(c + 1) * _CHUNK, :],
                x_ref[:, :],
                preferred_element_type=jnp.float32,
            )

            @pl.when(pred(c))
            def _(c=c):
                chunk_rdma(c).start()

        for c in range(n_chunks):
            @pl.when(pred(c))
            def _(c=c):
                chunk_rdma(c).wait()

        rows = lax.broadcasted_iota(jnp.int32, (m, n), 0)
        sel = jnp.where(rows < nk_, xks_buf[:, :], recv_buf[:, :])
        shift = jnp.where(my_y == 0, 0, ns_)
        out_ref[:, :] = pltpu.roll(sel, shift, axis=0)

    return pl.pallas_call(
        body,
        out_shape=jax.ShapeDtypeStruct((m, n), x.dtype),
        in_specs=[
            pl.BlockSpec(memory_space=pltpu.SMEM),
            pl.BlockSpec(memory_space=pltpu.VMEM),
            pl.BlockSpec(memory_space=pltpu.VMEM),
        ],
        out_specs=pl.BlockSpec(memory_space=pltpu.VMEM),
        scratch_shapes=[
            pltpu.VMEM((m, n), x.dtype),
            pltpu.VMEM((m, n), x.dtype),
            pltpu.SemaphoreType.DMA((n_chunks,)),
            pltpu.SemaphoreType.DMA((n_chunks,)),
        ],
        compiler_params=pltpu.CompilerParams(collective_id=0),
    )(cnt, dest.reshape(1, m), x)


# device time: 10329 ns/iter; 1.1414x vs baseline; 1.1414x over previous
import jax
import jax.numpy as jnp
from jax import lax
from jax.experimental import pallas as pl
from jax.experimental.pallas import tpu as pltpu

_CHUNK = 32


def kernel(x, dest):
    m, n = x.shape
    n_chunks = m // _CHUNK

    def body(dest_ref, x_ref, out_ref, xks_buf, recv_buf,
             send_sems, recv_sems):
        my_x = lax.axis_index("x")
        my_y = lax.axis_index("y")
        my_z = lax.axis_index("z")
        peer = (my_x, 1 - my_y, my_z)

        barrier_sem = pltpu.get_barrier_semaphore()
        pl.semaphore_signal(
            barrier_sem, inc=1, device_id=peer,
            device_id_type=pl.DeviceIdType.MESH,
        )
        pl.semaphore_wait(barrier_sem, 1)

        vf = (dest_ref[:, :] != my_y).astype(jnp.float32)
        lane = lax.broadcasted_iota(jnp.int32, (1, m), 1)
        cs = vf
        s = 1
        while s < m:
            cs = cs + jnp.where(lane >= s, pltpu.roll(cs, s, axis=1), 0.0)
            s *= 2
        ns_ = cs[0, m - 1].astype(jnp.int32)
        nk_ = m - ns_
        idxr = lane.astype(jnp.float32)
        pos = jnp.where(vf == 1.0, nk_.astype(jnp.float32) + cs - 1.0, idxr - cs)

        j_f = lax.broadcasted_iota(jnp.int32, (m, m), 0).astype(jnp.float32)
        p_mat = (j_f == pos).astype(jnp.float32)

        def chunk_rdma(c):
            return pltpu.make_async_remote_copy(
                src_ref=xks_buf.at[pl.ds(c * _CHUNK, _CHUNK)],
                dst_ref=recv_buf.at[pl.ds(c * _CHUNK, _CHUNK)],
                send_sem=send_sems.at[c],
                recv_sem=recv_sems.at[c],
                device_id=peer,
                device_id_type=pl.DeviceIdType.MESH,
            )

        def pred(c):
            if c == n_chunks - 1:
                return nk_ >= 0
            return (c + 1) * _CHUNK > nk_

        for c in reversed(range(n_chunks)):
            xks_buf[c * _CHUNK:(c + 1) * _CHUNK, :] = jax.lax.dot(
                p_mat[c * _CHUNK:(c + 1) * _CHUNK, :],
                x_ref[:, :],
                preferred_element_type=jnp.float32,
            )

            @pl.when(pred(c))
            def _(c=c):
                chunk_rdma(c).start()

        for c in range(n_chunks):
            @pl.when(pred(c))
            def _(c=c):
                chunk_rdma(c).wait()

        rows = lax.broadcasted_iota(jnp.int32, (m, n), 0)
        sel = jnp.where(rows < nk_, xks_buf[:, :], recv_buf[:, :])
        shift = jnp.where(my_y == 0, 0, ns_)
        out_ref[:, :] = pltpu.roll(sel, shift, axis=0)

    return pl.pallas_call(
        body,
        out_shape=jax.ShapeDtypeStruct((m, n), x.dtype),
        in_specs=[
            pl.BlockSpec(memory_space=pltpu.VMEM),
            pl.BlockSpec(memory_space=pltpu.VMEM),
        ],
        out_specs=pl.BlockSpec(memory_space=pltpu.VMEM),
        scratch_shapes=[
            pltpu.VMEM((m, n), x.dtype),
            pltpu.VMEM((m, n), x.dtype),
            pltpu.SemaphoreType.DMA((n_chunks,)),
            pltpu.SemaphoreType.DMA((n_chunks,)),
        ],
        compiler_params=pltpu.CompilerParams(collective_id=0),
    )(dest.reshape(1, m), x)


# device time: 9994 ns/iter; 1.1797x vs baseline; 1.0335x over previous
import jax
import jax.numpy as jnp
from jax import lax
from jax.experimental import pallas as pl
from jax.experimental.pallas import tpu as pltpu

_CHUNK = 32


def kernel(x, dest):
    m, n = x.shape
    n_chunks = m // _CHUNK

    def body(dest_ref, x_ref, out_ref, send_buf, recv_buf, dummy_buf,
             send_sems, recv_sems):
        my_x = lax.axis_index("x")
        my_y = lax.axis_index("y")
        my_z = lax.axis_index("z")
        peer = (my_x, 1 - my_y, my_z)
        is0 = my_y == 0

        barrier_sem = pltpu.get_barrier_semaphore()
        pl.semaphore_signal(
            barrier_sem, inc=1, device_id=peer,
            device_id_type=pl.DeviceIdType.MESH,
        )
        pl.semaphore_wait(barrier_sem, 1)

        vf = (dest_ref[:, :] != my_y).astype(jnp.float32)
        lane = lax.broadcasted_iota(jnp.int32, (1, m), 1)
        cs = vf
        s = 1
        while s < m:
            cs = cs + jnp.where(lane >= s, pltpu.roll(cs, s, axis=1), 0.0)
            s *= 2
        ns_ = cs[0, m - 1].astype(jnp.int32)
        nk_ = m - ns_
        idxr = lane.astype(jnp.float32)

        zero = jnp.float32(0.0)
        send_base = jnp.where(is0, zero, nk_.astype(jnp.float32))
        keep_base = jnp.where(is0, zero, ns_.astype(jnp.float32))
        pos_send = jnp.where(vf == 1.0, cs - 1.0 + send_base, -1.0)
        pos_keep = jnp.where(vf == 0.0, idxr - cs + keep_base, -1.0)

        def send_pred(c):
            return jnp.where(is0, c * _CHUNK < ns_, (c + 1) * _CHUNK > nk_)

        def recv_pred(c):
            return jnp.where(is0, (c + 1) * _CHUNK > nk_, c * _CHUNK < ns_)

        def keep_pred(c):
            return jnp.where(is0, c * _CHUNK < nk_, (c + 1) * _CHUNK > ns_)

        def chunk_iota_f(c):
            return (
                lax.broadcasted_iota(jnp.int32, (_CHUNK, m), 0) + c * _CHUNK
            ).astype(jnp.float32)

        def chunk_rdma(c):
            return pltpu.make_async_remote_copy(
                src_ref=send_buf.at[pl.ds(c * _CHUNK, _CHUNK)],
                dst_ref=recv_buf.at[pl.ds(c * _CHUNK, _CHUNK)],
                send_sem=send_sems.at[c],
                recv_sem=recv_sems.at[c],
                device_id=peer,
                device_id_type=pl.DeviceIdType.MESH,
            )

        def dummy_rdma():
            return pltpu.make_async_remote_copy(
                src_ref=send_buf.at[pl.ds(0, _CHUNK)],
                dst_ref=dummy_buf,
                send_sem=send_sems.at[n_chunks],
                recv_sem=recv_sems.at[n_chunks],
                device_id=peer,
                device_id_type=pl.DeviceIdType.MESH,
            )

        for c in range(n_chunks):
            @pl.when(send_pred(c))
            def _(c=c):
                p_chunk = (chunk_iota_f(c) == pos_send).astype(jnp.float32)
                send_buf[c * _CHUNK:(c + 1) * _CHUNK, :] = jax.lax.dot(
                    p_chunk, x_ref[:, :], preferred_element_type=jnp.float32
                )
                chunk_rdma(c).start()

        @pl.when(ns_ == 0)
        def _():
            dummy_rdma().start()

        for c in range(n_chunks):
            @pl.when(keep_pred(c))
            def _(c=c):
                p_chunk = (chunk_iota_f(c) == pos_keep).astype(jnp.float32)
                out_ref[c * _CHUNK:(c + 1) * _CHUNK, :] = jax.lax.dot(
                    p_chunk, x_ref[:, :], preferred_element_type=jnp.float32
                )

        for c in range(n_chunks):
            @pl.when(send_pred(c))
            def _(c=c):
                chunk_rdma(c).wait_send()

            @pl.when(recv_pred(c))
            def _(c=c):
                chunk_rdma(c).wait_recv()

            sl = pl.ds(c * _CHUNK, _CHUNK)

            @pl.when(jnp.logical_and(recv_pred(c), keep_pred(c)))
            def _(c=c, sl=sl):
                out_ref[sl, :] = out_ref[sl, :] + recv_buf[sl, :]

            @pl.when(jnp.logical_and(recv_pred(c), jnp.logical_not(keep_pred(c))))
            def _(c=c, sl=sl):
                out_ref[sl, :] = recv_buf[sl, :]

        @pl.when(ns_ == 0)
        def _():
            d = dummy_rdma()
            d.wait_send()
            d.wait_recv()

    return pl.pallas_call(
        body,
        out_shape=jax.ShapeDtypeStruct((m, n), x.dtype),
        in_specs=[
            pl.BlockSpec(memory_space=pltpu.VMEM),
            pl.BlockSpec(memory_space=pltpu.VMEM),
        ],
        out_specs=pl.BlockSpec(memory_space=pltpu.VMEM),
        scratch_shapes=[
            pltpu.VMEM((m, n), x.dtype),
            pltpu.VMEM((m, n), x.dtype),
            pltpu.VMEM((_CHUNK, n), x.dtype),
            pltpu.SemaphoreType.DMA((n_chunks + 1,)),
            pltpu.SemaphoreType.DMA((n_chunks + 1,)),
        ],
        compiler_params=pltpu.CompilerParams(collective_id=0),
    )(dest.reshape(1, m), x)
